# Initial kernel scaffold; baseline (speedup 1.0000x reference)
import jax
import jax.numpy as jnp
from jax.experimental import pallas as pl


def kernel(x, edge_index, W, a):
    raise NotImplementedError("write your pallas kernel here")


# speedup vs baseline: 8.5106x; 8.5106x over previous
import jax
import jax.numpy as jnp
from jax import lax
from jax.experimental import pallas as pl
from jax.experimental.pallas import tpu as pltpu
from jax.experimental.pallas import tpu_sc as plsc

N_NODES = 10000
N_EDGES = 320000
D = 128
DH = D // 2

NC = 2
NS = 16
EPT = N_EDGES // NS
C = 80
NCH = EPT // C
ZB = 1000
DZ = 2000


def _feat_body(x_ref, w_ref, a_ref, h_ref, s1_ref, s2_ref,
               m1_ref, m2_ref):
    h = jnp.dot(x_ref[...], w_ref[...], preferred_element_type=jnp.float32)
    h_ref[...] = h
    a = a_ref[...]
    s1 = jnp.dot(h, a[:D, :], preferred_element_type=jnp.float32)
    s2 = jnp.dot(h, a[D:, :], preferred_element_type=jnp.float32)
    s1_ref[...] = s1
    s2_ref[...] = s2

    @pl.when(pl.program_id(0) == 0)
    def _init():
        m1_ref[...] = jnp.full((1, D), -3.0e38, jnp.float32)
        m2_ref[...] = jnp.full((1, D), -3.0e38, jnp.float32)

    m1_ref[...] = jnp.maximum(m1_ref[...], jnp.max(s1))
    m2_ref[...] = jnp.maximum(m2_ref[...], jnp.max(s2))


def _features(x, W, a):
    blk = 1000
    return pl.pallas_call(
        _feat_body,
        grid=(N_NODES // blk,),
        in_specs=[
            pl.BlockSpec((blk, D), lambda i: (i, 0)),
            pl.BlockSpec((D, D), lambda i: (0, 0)),
            pl.BlockSpec((2 * D, 1), lambda i: (0, 0)),
        ],
        out_specs=[
            pl.BlockSpec((blk, D), lambda i: (i, 0)),
            pl.BlockSpec((blk, 1), lambda i: (i, 0)),
            pl.BlockSpec((blk, 1), lambda i: (i, 0)),
            pl.BlockSpec((1, D), lambda i: (0, 0)),
            pl.BlockSpec((1, D), lambda i: (0, 0)),
        ],
        out_shape=[
            jax.ShapeDtypeStruct((N_NODES, D), jnp.float32),
            jax.ShapeDtypeStruct((N_NODES, 1), jnp.float32),
            jax.ShapeDtypeStruct((N_NODES, 1), jnp.float32),
            jax.ShapeDtypeStruct((1, D), jnp.float32),
            jax.ShapeDtypeStruct((1, D), jnp.float32),
        ],
    )(x, W, a)


def _edge_body(h_hbm, s1_hbm, s2_hbm, row_hbm, col_hbm, c_hbm,
               num_out, den_out,
               s1_v, s2_v, row_v, col_v, exj_v, msg_v, msgh_v, zb_v, c_v,
               acc_sh, den_sh):
    cid = lax.axis_index("c")
    sid = lax.axis_index("s")

    pltpu.sync_copy(s1_hbm, s1_v)
    pltpu.sync_copy(s2_hbm, s2_v)
    pltpu.sync_copy(row_hbm.at[sid], row_v)
    pltpu.sync_copy(col_hbm.at[sid], col_v)
    pltpu.sync_copy(c_hbm, c_v)

    zeros16 = jnp.zeros((16,), jnp.float32)

    @pl.loop(0, C)
    def _zmsg(i):
        for q in range(DH // 16):
            msgh_v[i, pl.ds(q * 16, 16)] = zeros16

    @pl.loop(0, DZ // 16)
    def _zzb(i):
        zb_v[pl.ds(i * 16, 16)] = zeros16

    @pl.when(sid < N_NODES // ZB)
    def _zacc():
        zbase = sid * ZB
        for k in range(12):
            pltpu.sync_copy(msgh_v, acc_sh.at[pl.ds(zbase + k * C, C)])
        pltpu.sync_copy(msgh_v, acc_sh.at[pl.ds(zbase + ZB - C, C)])

    @pl.when(sid < N_NODES // DZ)
    def _zden():
        pltpu.sync_copy(zb_v, den_sh.at[pl.ds(sid * DZ, DZ)])

    plsc.subcore_barrier()

    cvec = c_v[...]

    def _exp_np(y):
        t = y * 1.4426950408889634
        k = jnp.maximum(t.astype(jnp.int32), -126)
        r = (t - k.astype(jnp.float32)) * 0.6931471805599453
        p = jnp.ones((16,), jnp.float32)
        for i in range(7, 0, -1):
            p = 1.0 + p * r * (1.0 / i)
        scale = plsc.bitcast((k + 127) << 23, jnp.float32)
        return p * scale

    def _main_loop(col_base):
        @pl.loop(0, NCH)
        def _chunk(j):
            for k in range(C // 16):
                r16 = row_v[j, pl.ds(k * 16, 16)]
                c16 = col_v[j, pl.ds(k * 16, 16)]
                z = (plsc.load_gather(s1_v, [r16])
                     + plsc.load_gather(s2_v, [c16]))
                e = jnp.where(z >= 0.0, z, 0.2 * z)
                exj_v[pl.ds(k * 16, 16)] = _exp_np(e - cvec)

            pltpu.sync_copy(h_hbm.at[col_v.at[j]], msg_v)

            @pl.loop(0, C)
            def _srow(i):
                av = plsc.load_gather(exj_v, [jnp.full((16,), i, jnp.int32)])
                for q in range(DH // 16):
                    msgh_v[i, pl.ds(q * 16, 16)] = (
                        msg_v[i, pl.ds(col_base + q * 16, 16)] * av)

            pltpu.sync_copy(msgh_v, acc_sh.at[row_v.at[j]], add=True)
            pltpu.sync_copy(exj_v, den_sh.at[row_v.at[j]], add=True)

    @pl.when(cid == 0)
    def _run0():
        _main_loop(0)

    @pl.when(cid == 1)
    def _run1():
        _main_loop(DH)

    plsc.subcore_barrier()

    @pl.when(sid < N_NODES // ZB)
    def _wout():
        zbase = sid * ZB
        pltpu.sync_copy(acc_sh.at[pl.ds(zbase, ZB)],
                        num_out.at[cid, pl.ds(zbase, ZB)])

    @pl.when(sid < N_NODES // DZ)
    def _wden():
        dbase = sid * DZ
        pltpu.sync_copy(den_sh.at[pl.ds(dbase, DZ)], zb_v)
        pltpu.sync_copy(zb_v, den_out.at[pl.ds(cid * N_NODES + dbase, DZ)])


def _edge_aggregate(h, s1, s2, row3, col3, c16):
    mesh = plsc.VectorSubcoreMesh(core_axis_name="c", subcore_axis_name="s",
                                  num_cores=NC, num_subcores=NS)
    f = pl.kernel(
        _edge_body,
        out_type=[
            jax.ShapeDtypeStruct((NC, N_NODES, DH), jnp.float32),
            jax.ShapeDtypeStruct((NC * N_NODES,), jnp.float32),
        ],
        mesh=mesh,
        compiler_params=pltpu.CompilerParams(needs_layout_passes=False,
                                             use_tc_tiling_on_sc=False),
        scratch_types=[
            pltpu.VMEM((N_NODES,), jnp.float32),
            pltpu.VMEM((N_NODES,), jnp.float32),
            pltpu.VMEM((NCH, C), jnp.int32),
            pltpu.VMEM((NCH, C), jnp.int32),
            pltpu.VMEM((C,), jnp.float32),
            pltpu.VMEM((C, D), jnp.float32),
            pltpu.VMEM((C, DH), jnp.float32),
            pltpu.VMEM((DZ,), jnp.float32),
            pltpu.VMEM((16,), jnp.float32),
            pltpu.VMEM_SHARED((N_NODES, DH), jnp.float32),
            pltpu.VMEM_SHARED((N_NODES,), jnp.float32),
        ],
    )
    return f(h, s1, s2, row3, col3, c16)


def _combine_body(n0_ref, n1_ref, d_ref, o_ref):
    d = d_ref[...]
    d = jnp.where(d > 0.0, d, 1.0)
    o = jnp.concatenate([n0_ref[...], n1_ref[...]], axis=1) / d
    o_ref[...] = jnp.maximum(o, 0.0)


def _combine(n0, n1, den):
    blk = 1000
    return pl.pallas_call(
        _combine_body,
        grid=(N_NODES // blk,),
        in_specs=[
            pl.BlockSpec((blk, DH), lambda i: (i, 0)),
            pl.BlockSpec((blk, DH), lambda i: (i, 0)),
            pl.BlockSpec((blk, 1), lambda i: (i, 0)),
        ],
        out_specs=pl.BlockSpec((blk, D), lambda i: (i, 0)),
        out_shape=jax.ShapeDtypeStruct((N_NODES, D), jnp.float32),
    )(n0, n1, den)


@jax.jit
def kernel(x, edge_index, W, a):
    h, s1, s2, m1, m2 = _features(x, W, a)
    zmax = m1[0, 0] + m2[0, 0]
    cshift = jnp.where(zmax >= 0.0, zmax, 0.2 * zmax)
    c16 = jnp.full((16,), cshift, jnp.float32)
    row3 = edge_index[0].reshape(NS, NCH, C)
    col3 = edge_index[1].reshape(NS, NCH, C)
    num, den = _edge_aggregate(h, s1.reshape(-1), s2.reshape(-1),
                               row3, col3, c16)
    return _combine(num[0], num[1], den[:N_NODES].reshape(-1, 1))


# speedup vs baseline: 13.0905x; 1.5382x over previous
import jax
import jax.numpy as jnp
from jax import lax
from jax.experimental import pallas as pl
from jax.experimental.pallas import tpu as pltpu
from jax.experimental.pallas import tpu_sc as plsc

N_NODES = 10000
N_EDGES = 320000
D = 128
DH = D // 2

NC = 2
NS = 16
EPT = N_EDGES // NS
C = 80
NCH = EPT // C
ZB = 1000
DZ = 2000


def _feat_body(x_ref, w_ref, a_ref, h_ref, s1_ref, s2_ref, m1_ref, m2_ref):
    h = jnp.dot(x_ref[...], w_ref[...], preferred_element_type=jnp.float32)
    h_ref[...] = h
    a = a_ref[...]
    s1 = jnp.dot(h, a[:D, :], preferred_element_type=jnp.float32)
    s2 = jnp.dot(h, a[D:, :], preferred_element_type=jnp.float32)
    s1_ref[...] = s1
    s2_ref[...] = s2

    @pl.when(pl.program_id(0) == 0)
    def _init():
        m1_ref[...] = jnp.full((1, D), -3.0e38, jnp.float32)
        m2_ref[...] = jnp.full((1, D), -3.0e38, jnp.float32)

    m1_ref[...] = jnp.maximum(m1_ref[...], jnp.max(s1))
    m2_ref[...] = jnp.maximum(m2_ref[...], jnp.max(s2))


def _features(x, W, a):
    blk = 1000
    return pl.pallas_call(
        _feat_body,
        grid=(N_NODES // blk,),
        in_specs=[
            pl.BlockSpec((blk, D), lambda i: (i, 0)),
            pl.BlockSpec((D, D), lambda i: (0, 0)),
            pl.BlockSpec((2 * D, 1), lambda i: (0, 0)),
        ],
        out_specs=[
            pl.BlockSpec((blk, D), lambda i: (i, 0)),
            pl.BlockSpec((blk, 1), lambda i: (i, 0)),
            pl.BlockSpec((blk, 1), lambda i: (i, 0)),
            pl.BlockSpec((1, D), lambda i: (0, 0)),
            pl.BlockSpec((1, D), lambda i: (0, 0)),
        ],
        out_shape=[
            jax.ShapeDtypeStruct((N_NODES, D), jnp.float32),
            jax.ShapeDtypeStruct((N_NODES, 1), jnp.float32),
            jax.ShapeDtypeStruct((N_NODES, 1), jnp.float32),
            jax.ShapeDtypeStruct((1, D), jnp.float32),
            jax.ShapeDtypeStruct((1, D), jnp.float32),
        ],
    )(x, W, a)


def _edge_body(h_hbm, s1_hbm, s2_hbm, rc_hbm, c_hbm,
               num_out, den_out,
               s1_v, s2_v, rc_v, exj_v, msg_v, msgh_v, zb_v, c_v,
               acc_sh, den_sh,
               g_sem, sa_sem, sd_sem, rc_sem):
    cid = lax.axis_index("c")
    sid = lax.axis_index("s")

    pltpu.sync_copy(s1_hbm, s1_v)
    pltpu.sync_copy(s2_hbm, s2_v)
    pltpu.sync_copy(c_hbm, c_v)

    zeros16 = jnp.zeros((16,), jnp.float32)

    @pl.loop(0, C)
    def _zmsg(i):
        for q in range(DH // 16):
            msgh_v[0, i, pl.ds(q * 16, 16)] = zeros16

    @pl.loop(0, DZ // 16)
    def _zzb(i):
        zb_v[pl.ds(i * 16, 16)] = zeros16

    @pl.when(sid < N_NODES // ZB)
    def _zacc():
        zbase = sid * ZB
        for k in range(12):
            pltpu.sync_copy(msgh_v.at[0], acc_sh.at[pl.ds(zbase + k * C, C)])
        pltpu.sync_copy(msgh_v.at[0], acc_sh.at[pl.ds(zbase + ZB - C, C)])

    @pl.when(sid < N_NODES // DZ)
    def _zden():
        pltpu.sync_copy(zb_v, den_sh.at[pl.ds(sid * DZ, DZ)])

    plsc.subcore_barrier()

    cvec = c_v[...]

    def _exp_np(y):
        t = y * 1.4426950408889634
        k = jnp.maximum(t.astype(jnp.int32), -126)
        r = (t - k.astype(jnp.float32)) * 0.6931471805599453
        p = jnp.ones((16,), jnp.float32)
        for i in range(7, 0, -1):
            p = 1.0 + p * r * (1.0 / i)
        scale = plsc.bitcast((k + 127) << 23, jnp.float32)
        return p * scale

    def _main_loop(col_base):
        def _rc_fetch(j):
            pltpu.async_copy(rc_hbm.at[sid, j], rc_v.at[j & 3],
                             rc_sem.at[j & 3])

        def _rc_wait(j):
            pltpu.make_async_copy(rc_hbm.at[sid, j], rc_v.at[j & 3],
                                  rc_sem.at[j & 3]).wait()

        def _gather_start(j, b):
            pltpu.async_copy(h_hbm.at[rc_v.at[j & 3, 1]], msg_v.at[b],
                             g_sem.at[b])

        def _chunk(j, b):
            g = j & 3

            @pl.when(j >= 2)
            def _drain():
                pltpu.make_async_copy(msgh_v.at[b],
                                      acc_sh.at[rc_v.at[g, 0]],
                                      sa_sem.at[b]).wait()
                pltpu.make_async_copy(exj_v.at[b],
                                      den_sh.at[rc_v.at[g, 0]],
                                      sd_sem.at[b]).wait()

            @pl.when(j + 2 < NCH)
            def _fnext():
                _rc_fetch(j + 2)

            @pl.when(j + 1 < NCH)
            def _gnext():
                _rc_wait(j + 1)
                _gather_start(j + 1, 1 - b)

            for k in range(C // 16):
                r16 = rc_v[g, 0, pl.ds(k * 16, 16)]
                c16 = rc_v[g, 1, pl.ds(k * 16, 16)]
                z = (plsc.load_gather(s1_v, [r16])
                     + plsc.load_gather(s2_v, [c16]))
                e = jnp.where(z >= 0.0, z, 0.2 * z)
                exj_v[b, pl.ds(k * 16, 16)] = _exp_np(e - cvec)

            pltpu.make_async_copy(h_hbm.at[rc_v.at[g, 1]], msg_v.at[b],
                                  g_sem.at[b]).wait()

            @pl.loop(0, C)
            def _srow(i):
                av = plsc.load_gather(exj_v.at[b],
                                      [jnp.full((16,), i, jnp.int32)])
                for q in range(DH // 16):
                    msgh_v[b, i, pl.ds(q * 16, 16)] = (
                        msg_v[b, i, pl.ds(col_base + q * 16, 16)] * av)

            pltpu.async_copy(msgh_v.at[b], acc_sh.at[rc_v.at[g, 0]],
                             sa_sem.at[b], add=True)
            pltpu.async_copy(exj_v.at[b], den_sh.at[rc_v.at[g, 0]],
                             sd_sem.at[b], add=True)

        _rc_fetch(0)
        _rc_fetch(1)
        _rc_wait(0)
        _gather_start(0, 0)

        @pl.loop(0, NCH // 2)
        def _step(s):
            _chunk(s * 2, 0)
            _chunk(s * 2 + 1, 1)

        for b in range(2):
            jl = NCH - 2 + b
            pltpu.make_async_copy(msgh_v.at[b], acc_sh.at[rc_v.at[jl & 3, 0]],
                                  sa_sem.at[b]).wait()
            pltpu.make_async_copy(exj_v.at[b], den_sh.at[rc_v.at[jl & 3, 0]],
                                  sd_sem.at[b]).wait()

    @pl.when(cid == 0)
    def _run0():
        _main_loop(0)

    @pl.when(cid == 1)
    def _run1():
        _main_loop(DH)

    plsc.subcore_barrier()

    @pl.when(sid < N_NODES // ZB)
    def _wout():
        zbase = sid * ZB
        pltpu.sync_copy(acc_sh.at[pl.ds(zbase, ZB)],
                        num_out.at[cid, pl.ds(zbase, ZB)])

    @pl.when(sid < N_NODES // DZ)
    def _wden():
        dbase = sid * DZ
        pltpu.sync_copy(den_sh.at[pl.ds(dbase, DZ)], zb_v)
        pltpu.sync_copy(zb_v, den_out.at[pl.ds(cid * N_NODES + dbase, DZ)])


def _edge_aggregate(h, s1, s2, rc4, c16):
    mesh = plsc.VectorSubcoreMesh(core_axis_name="c", subcore_axis_name="s",
                                  num_cores=NC, num_subcores=NS)
    f = pl.kernel(
        _edge_body,
        out_type=[
            jax.ShapeDtypeStruct((NC, N_NODES, DH), jnp.float32),
            jax.ShapeDtypeStruct((NC * N_NODES,), jnp.float32),
        ],
        mesh=mesh,
        compiler_params=pltpu.CompilerParams(needs_layout_passes=False,
                                             use_tc_tiling_on_sc=False),
        scratch_types=[
            pltpu.VMEM((N_NODES,), jnp.float32),
            pltpu.VMEM((N_NODES,), jnp.float32),
            pltpu.VMEM((4, 2, C), jnp.int32),
            pltpu.VMEM((2, C), jnp.float32),
            pltpu.VMEM((2, C, D), jnp.float32),
            pltpu.VMEM((2, C, DH), jnp.float32),
            pltpu.VMEM((DZ,), jnp.float32),
            pltpu.VMEM((16,), jnp.float32),
            pltpu.VMEM_SHARED((N_NODES, DH), jnp.float32),
            pltpu.VMEM_SHARED((N_NODES,), jnp.float32),
            pltpu.SemaphoreType.DMA((2,)),
            pltpu.SemaphoreType.DMA((2,)),
            pltpu.SemaphoreType.DMA((2,)),
            pltpu.SemaphoreType.DMA((4,)),
        ],
    )
    return f(h, s1, s2, rc4, c16)


def _combine_body(n0_ref, n1_ref, d_ref, o_ref):
    d = d_ref[...]
    d = jnp.where(d > 0.0, d, 1.0)
    o = jnp.concatenate([n0_ref[...], n1_ref[...]], axis=1) / d
    o_ref[...] = jnp.maximum(o, 0.0)


def _combine(n0, n1, den):
    blk = 1000
    return pl.pallas_call(
        _combine_body,
        grid=(N_NODES // blk,),
        in_specs=[
            pl.BlockSpec((blk, DH), lambda i: (i, 0)),
            pl.BlockSpec((blk, DH), lambda i: (i, 0)),
            pl.BlockSpec((blk, 1), lambda i: (i, 0)),
        ],
        out_specs=pl.BlockSpec((blk, D), lambda i: (i, 0)),
        out_shape=jax.ShapeDtypeStruct((N_NODES, D), jnp.float32),
    )(n0, n1, den)


@jax.jit
def kernel(x, edge_index, W, a):
    h, s1, s2, m1, m2 = _features(x, W, a)
    zmax = m1[0, 0] + m2[0, 0]
    cshift = jnp.where(zmax >= 0.0, zmax, 0.2 * zmax)
    c16 = jnp.full((16,), cshift, jnp.float32)
    rc4 = jnp.stack([edge_index[0].reshape(NS, NCH, C),
                     edge_index[1].reshape(NS, NCH, C)], axis=2)
    num, den = _edge_aggregate(h, s1.reshape(-1), s2.reshape(-1), rc4, c16)
    den0 = den[:N_NODES]
    return _combine(num[0], num[1], den0.reshape(-1, 1))


# speedup vs baseline: 22.4399x; 1.7142x over previous
import jax
import jax.numpy as jnp
from jax import lax
from jax.experimental import pallas as pl
from jax.experimental.pallas import tpu as pltpu
from jax.experimental.pallas import tpu_sc as plsc

N_NODES = 10000
N_EDGES = 320000
D = 128
DH = D // 2

NC = 2
NS = 16
EPT = N_EDGES // NS
C = 80
NCH = EPT // C
ZB = 1000
DZ = 2000


def _feat_body(x_ref, w_ref, a_ref, h0_ref, h1_ref, s1_ref, s2_ref,
               m1_ref, m2_ref):
    h = jnp.dot(x_ref[...], w_ref[...], preferred_element_type=jnp.float32)
    h0_ref[...] = h[:, :DH]
    h1_ref[...] = h[:, DH:]
    a = a_ref[...]
    s1 = jnp.dot(h, a[:D, :], preferred_element_type=jnp.float32)
    s2 = jnp.dot(h, a[D:, :], preferred_element_type=jnp.float32)
    s1_ref[...] = s1
    s2_ref[...] = s2

    @pl.when(pl.program_id(0) == 0)
    def _init():
        m1_ref[...] = jnp.full((1, D), -3.0e38, jnp.float32)
        m2_ref[...] = jnp.full((1, D), -3.0e38, jnp.float32)

    m1_ref[...] = jnp.maximum(m1_ref[...], jnp.max(s1))
    m2_ref[...] = jnp.maximum(m2_ref[...], jnp.max(s2))


def _features(x, W, a):
    blk = 1000
    return pl.pallas_call(
        _feat_body,
        grid=(N_NODES // blk,),
        in_specs=[
            pl.BlockSpec((blk, D), lambda i: (i, 0)),
            pl.BlockSpec((D, D), lambda i: (0, 0)),
            pl.BlockSpec((2 * D, 1), lambda i: (0, 0)),
        ],
        out_specs=[
            pl.BlockSpec((blk, DH), lambda i: (i, 0)),
            pl.BlockSpec((blk, DH), lambda i: (i, 0)),
            pl.BlockSpec((blk, 1), lambda i: (i, 0)),
            pl.BlockSpec((blk, 1), lambda i: (i, 0)),
            pl.BlockSpec((1, D), lambda i: (0, 0)),
            pl.BlockSpec((1, D), lambda i: (0, 0)),
        ],
        out_shape=[
            jax.ShapeDtypeStruct((N_NODES, DH), jnp.float32),
            jax.ShapeDtypeStruct((N_NODES, DH), jnp.float32),
            jax.ShapeDtypeStruct((N_NODES, 1), jnp.float32),
            jax.ShapeDtypeStruct((N_NODES, 1), jnp.float32),
            jax.ShapeDtypeStruct((1, D), jnp.float32),
            jax.ShapeDtypeStruct((1, D), jnp.float32),
        ],
    )(x, W, a)


def _edge_body(h0_hbm, h1_hbm, s1_hbm, s2_hbm, rc_hbm, c_hbm,
               num_out, den_out,
               s1_v, s2_v, rc_v, exj_v, msg_v, zb_v, c_v,
               acc_sh, den_sh,
               g_sem, sa_sem, sd_sem, rc_sem):
    cid = lax.axis_index("c")
    sid = lax.axis_index("s")

    pltpu.sync_copy(s1_hbm, s1_v)
    pltpu.sync_copy(s2_hbm, s2_v)
    pltpu.sync_copy(c_hbm, c_v)

    zeros16 = jnp.zeros((16,), jnp.float32)

    @pl.loop(0, C)
    def _zmsg(i):
        for q in range(DH // 16):
            msg_v[0, i, pl.ds(q * 16, 16)] = zeros16

    @pl.loop(0, DZ // 16)
    def _zzb(i):
        zb_v[pl.ds(i * 16, 16)] = zeros16

    @pl.when(sid < N_NODES // ZB)
    def _zacc():
        zbase = sid * ZB
        for k in range(12):
            pltpu.sync_copy(msg_v.at[0], acc_sh.at[pl.ds(zbase + k * C, C)])
        pltpu.sync_copy(msg_v.at[0], acc_sh.at[pl.ds(zbase + ZB - C, C)])

    @pl.when(sid < N_NODES // DZ)
    def _zden():
        pltpu.sync_copy(zb_v, den_sh.at[pl.ds(sid * DZ, DZ)])

    plsc.subcore_barrier()

    cvec = c_v[...]

    def _exp_np(y):
        t = y * 1.4426950408889634
        k = jnp.maximum(t.astype(jnp.int32), -126)
        r = (t - k.astype(jnp.float32)) * 0.6931471805599453
        p = jnp.ones((16,), jnp.float32)
        for i in range(7, 0, -1):
            p = 1.0 + p * r * (1.0 / i)
        scale = plsc.bitcast((k + 127) << 23, jnp.float32)
        return p * scale

    def _main_loop(h_hbm):
        def _rc_fetch(j):
            pltpu.async_copy(rc_hbm.at[sid, j], rc_v.at[j & 3],
                             rc_sem.at[j & 3])

        def _rc_wait(j):
            pltpu.make_async_copy(rc_hbm.at[sid, j], rc_v.at[j & 3],
                                  rc_sem.at[j & 3]).wait()

        def _gather_start(j, b):
            pltpu.async_copy(h_hbm.at[rc_v.at[j & 3, 1]], msg_v.at[b],
                             g_sem.at[b])

        def _chunk(j, b):
            g = j & 3

            @pl.when(j >= 2)
            def _drain():
                pltpu.make_async_copy(msg_v.at[b],
                                      acc_sh.at[rc_v.at[g, 0]],
                                      sa_sem.at[b]).wait()
                pltpu.make_async_copy(exj_v.at[b],
                                      den_sh.at[rc_v.at[g, 0]],
                                      sd_sem.at[b]).wait()

            @pl.when(j + 2 < NCH)
            def _fnext():
                _rc_fetch(j + 2)

            @pl.when(j + 1 < NCH)
            def _gnext():
                _rc_wait(j + 1)
                _gather_start(j + 1, 1 - b)

            for k in range(C // 16):
                r16 = rc_v[g, 0, pl.ds(k * 16, 16)]
                c16 = rc_v[g, 1, pl.ds(k * 16, 16)]
                z = (plsc.load_gather(s1_v, [r16])
                     + plsc.load_gather(s2_v, [c16]))
                e = jnp.where(z >= 0.0, z, 0.2 * z)
                exj_v[b, pl.ds(k * 16, 16)] = _exp_np(e - cvec)

            pltpu.make_async_copy(h_hbm.at[rc_v.at[g, 1]], msg_v.at[b],
                                  g_sem.at[b]).wait()

            @pl.loop(0, C, unroll=4)
            def _srow(i):
                av = plsc.load_gather(exj_v.at[b],
                                      [jnp.full((16,), i, jnp.int32)])
                for q in range(DH // 16):
                    msg_v[b, i, pl.ds(q * 16, 16)] = (
                        msg_v[b, i, pl.ds(q * 16, 16)] * av)

            pltpu.async_copy(msg_v.at[b], acc_sh.at[rc_v.at[g, 0]],
                             sa_sem.at[b], add=True)
            pltpu.async_copy(exj_v.at[b], den_sh.at[rc_v.at[g, 0]],
                             sd_sem.at[b], add=True)

        _rc_fetch(0)
        _rc_fetch(1)
        _rc_wait(0)
        _gather_start(0, 0)

        @pl.loop(0, NCH // 2)
        def _step(s):
            _chunk(s * 2, 0)
            _chunk(s * 2 + 1, 1)

        for b in range(2):
            jl = NCH - 2 + b
            pltpu.make_async_copy(msg_v.at[b], acc_sh.at[rc_v.at[jl & 3, 0]],
                                  sa_sem.at[b]).wait()
            pltpu.make_async_copy(exj_v.at[b], den_sh.at[rc_v.at[jl & 3, 0]],
                                  sd_sem.at[b]).wait()

    @pl.when(cid == 0)
    def _run0():
        _main_loop(h0_hbm)

    @pl.when(cid == 1)
    def _run1():
        _main_loop(h1_hbm)

    plsc.subcore_barrier()

    @pl.when(sid < N_NODES // ZB)
    def _wout():
        zbase = sid * ZB
        pltpu.sync_copy(acc_sh.at[pl.ds(zbase, ZB)],
                        num_out.at[cid, pl.ds(zbase, ZB)])

    @pl.when(sid < N_NODES // DZ)
    def _wden():
        dbase = sid * DZ
        pltpu.sync_copy(den_sh.at[pl.ds(dbase, DZ)], zb_v)
        pltpu.sync_copy(zb_v, den_out.at[pl.ds(cid * N_NODES + dbase, DZ)])


def _edge_aggregate(h0, h1, s1, s2, rc4, c16):
    mesh = plsc.VectorSubcoreMesh(core_axis_name="c", subcore_axis_name="s",
                                  num_cores=NC, num_subcores=NS)
    f = pl.kernel(
        _edge_body,
        out_type=[
            jax.ShapeDtypeStruct((NC, N_NODES, DH), jnp.float32),
            jax.ShapeDtypeStruct((NC * N_NODES,), jnp.float32),
        ],
        mesh=mesh,
        compiler_params=pltpu.CompilerParams(needs_layout_passes=False,
                                             use_tc_tiling_on_sc=False),
        scratch_types=[
            pltpu.VMEM((N_NODES,), jnp.float32),
            pltpu.VMEM((N_NODES,), jnp.float32),
            pltpu.VMEM((4, 2, C), jnp.int32),
            pltpu.VMEM((2, C), jnp.float32),
            pltpu.VMEM((2, C, DH), jnp.float32),
            pltpu.VMEM((DZ,), jnp.float32),
            pltpu.VMEM((16,), jnp.float32),
            pltpu.VMEM_SHARED((N_NODES, DH), jnp.float32),
            pltpu.VMEM_SHARED((N_NODES,), jnp.float32),
            pltpu.SemaphoreType.DMA((2,)),
            pltpu.SemaphoreType.DMA((2,)),
            pltpu.SemaphoreType.DMA((2,)),
            pltpu.SemaphoreType.DMA((4,)),
        ],
    )
    return f(h0, h1, s1, s2, rc4, c16)


def _combine_body(n0_ref, n1_ref, d_ref, o_ref):
    d = d_ref[...]
    d = jnp.where(d > 0.0, d, 1.0)
    o = jnp.concatenate([n0_ref[...], n1_ref[...]], axis=1) / d
    o_ref[...] = jnp.maximum(o, 0.0)


def _combine(n0, n1, den):
    blk = 1000
    return pl.pallas_call(
        _combine_body,
        grid=(N_NODES // blk,),
        in_specs=[
            pl.BlockSpec((blk, DH), lambda i: (i, 0)),
            pl.BlockSpec((blk, DH), lambda i: (i, 0)),
            pl.BlockSpec((blk, 1), lambda i: (i, 0)),
        ],
        out_specs=pl.BlockSpec((blk, D), lambda i: (i, 0)),
        out_shape=jax.ShapeDtypeStruct((N_NODES, D), jnp.float32),
    )(n0, n1, den)


@jax.jit
def kernel(x, edge_index, W, a):
    h0, h1, s1, s2, m1, m2 = _features(x, W, a)
    zmax = m1[0, 0] + m2[0, 0]
    cshift = jnp.where(zmax >= 0.0, zmax, 0.2 * zmax)
    c16 = jnp.full((16,), cshift, jnp.float32)
    rc4 = jnp.stack([edge_index[0].reshape(NS, NCH, C),
                     edge_index[1].reshape(NS, NCH, C)], axis=2)
    num, den = _edge_aggregate(h0, h1, s1.reshape(-1), s2.reshape(-1),
                               rc4, c16)
    den0 = den[:N_NODES]
    return _combine(num[0], num[1], den0.reshape(-1, 1))
